# baseline (device time: 24636 ns/iter reference)
import jax
import jax.numpy as jnp
from jax import lax
from jax.experimental import pallas as pl
from jax.experimental.pallas import tpu as pltpu

ZDIM = 4
B, H, D, BS = 16, 16, 64, 16
NBT = 128
HG = H // 4
LANES = 128
NEG = -1e30
CHUNK = 4
NC = BS // CHUNK
XY_OFFS = ((0, 1), (1, 0), (1, 1))


def kernel(Q, K, V, bt, lens):
    nloc = K.shape[0]

    gx = lax.axis_index("x")
    gy = lax.axis_index("y")
    g = 2 * gx + gy

    q_hbd = Q.reshape(B, H, D).transpose(1, 0, 2)
    qg = lax.dynamic_slice_in_dim(q_hbd, g * HG, HG, 0)
    kt = K.transpose(1, 2, 3, 0)
    vt = V.transpose(1, 2, 3, 0)
    kg = lax.dynamic_slice_in_dim(kt, g * HG, HG, 1)
    vg = lax.dynamic_slice_in_dim(vt, g * HG, HG, 1)

    def body(q_ref, k_ref, v_ref, bt_ref, lens_ref, out_ref,
             commz_ref, commxy_ref, cnt_ref, mr_ref, lr_ref, or_ref,
             res_ref, zs_sems, zr_sems, xs_sems, xr_sems):
        c = pl.program_id(0)
        my_x = lax.axis_index("x")
        my_y = lax.axis_index("y")
        my_z = lax.axis_index("z")
        base = my_z * nloc
        barrier = pltpu.get_barrier_semaphore()

        @pl.when(c == 0)
        def _():
            with jax.named_scope("barrier_signal"):
                for dz in range(1, ZDIM):
                    pl.semaphore_signal(
                        barrier, inc=1,
                        device_id=(my_x, my_y, (my_z + dz) % ZDIM),
                        device_id_type=pl.DeviceIdType.MESH,
                    )
                for dx, dy in XY_OFFS:
                    pl.semaphore_signal(
                        barrier, inc=1,
                        device_id=((my_x + dx) % 2, (my_y + dy) % 2, my_z),
                        device_id_type=pl.DeviceIdType.MESH,
                    )
            with jax.named_scope("count"):
                btv = bt_ref[...]
                lensv = lens_ref[...]
                jidx = lax.broadcasted_iota(jnp.int32, (B, NBT, nloc), 1)
                pidx = lax.broadcasted_iota(jnp.int32, (B, NBT, nloc), 2)
                hits = ((btv[:, :, None] == base + pidx)
                        & (jidx < lensv[:, :, None]))
                cnt_ref[...] = jnp.sum(hits.astype(jnp.float32), axis=1)
            mr_ref[...] = jnp.full((HG, B), NEG, jnp.float32)
            lr_ref[...] = jnp.zeros((HG, B), jnp.float32)
            or_ref[...] = jnp.zeros((HG, B, D), jnp.float32)

        with jax.named_scope("attn_chunk"):
            cm = cnt_ref[...][None]
            q_all = q_ref[...]
            s_list = []
            for t in range(CHUNK):
                s_t = lax.dot_general(
                    q_all, k_ref[t], (((2,), (1,)), ((0,), (0,))),
                    preferred_element_type=jnp.float32,
                ) * (D ** -0.5)
                s_list.append(jnp.where(cm > 0, s_t, NEG))
            mc = jnp.max(s_list[0], axis=2)
            for s_t in s_list[1:]:
                mc = jnp.maximum(mc, jnp.max(s_t, axis=2))
            m_old = mr_ref[...]
            m_new = jnp.maximum(m_old, mc)
            scale = jnp.exp(m_old - m_new)
            l_acc = lr_ref[...] * scale
            o_acc = or_ref[...] * scale[:, :, None]
            for t in range(CHUNK):
                e_t = jnp.exp(s_list[t] - m_new[:, :, None]) * cm
                l_acc = l_acc + jnp.sum(e_t, axis=2)
                o_acc = o_acc + lax.dot_general(
                    e_t, v_ref[t], (((2,), (2,)), ((0,), (0,))),
                    preferred_element_type=jnp.float32,
                )
            mr_ref[...] = m_new
            lr_ref[...] = l_acc
            or_ref[...] = o_acc

        @pl.when(c == NC - 1)
        def _():
            with jax.named_scope("pack_z"):
                commz_ref[0, :, :, 0:D] = or_ref[...]
                commz_ref[0, :, :, D:D + 1] = mr_ref[...][:, :, None]
                commz_ref[0, :, :, D + 1:D + 2] = lr_ref[...][:, :, None]

            with jax.named_scope("barrier_wait"):
                pl.semaphore_wait(barrier, ZDIM - 1 + len(XY_OFFS))

            with jax.named_scope("z_a2a"):
                rdmas = []
                for dz in range(1, ZDIM):
                    rdma = pltpu.make_async_remote_copy(
                        src_ref=commz_ref.at[0],
                        dst_ref=commz_ref.at[dz],
                        send_sem=zs_sems.at[dz - 1],
                        recv_sem=zr_sems.at[dz - 1],
                        device_id=(my_x, my_y, (my_z + dz) % ZDIM),
                        device_id_type=pl.DeviceIdType.MESH,
                    )
                    rdma.start()
                    rdmas.append(rdma)
                for rdma in rdmas:
                    rdma.wait()

            with jax.named_scope("merge_z"):
                ms = [commz_ref[i, :, :, D:D + 1] for i in range(ZDIM)]
                mx = ms[0]
                for mi in ms[1:]:
                    mx = jnp.maximum(mx, mi)
                acc_o = jnp.zeros((HG, B, D), jnp.float32)
                acc_l = jnp.zeros((HG, B, 1), jnp.float32)
                for i in range(ZDIM):
                    alpha = jnp.exp(ms[i] - mx)
                    acc_o = acc_o + commz_ref[i, :, :, 0:D] * alpha
                    acc_l = acc_l + commz_ref[i, :, :, D + 1:D + 2] * alpha
                commxy_ref[0, :, :, 0:D] = acc_o / acc_l

            with jax.named_scope("xy_a2a"):
                rdmas = []
                for j, (dx, dy) in enumerate(XY_OFFS):
                    rdma = pltpu.make_async_remote_copy(
                        src_ref=commxy_ref.at[0],
                        dst_ref=commxy_ref.at[j + 1],
                        send_sem=xs_sems.at[j],
                        recv_sem=xr_sems.at[j],
                        device_id=((my_x + dx) % 2, (my_y + dy) % 2, my_z),
                        device_id_type=pl.DeviceIdType.MESH,
                    )
                    rdma.start()
                    rdmas.append(rdma)
                for rdma in rdmas:
                    rdma.wait()

            with jax.named_scope("assemble"):
                my_g = 2 * my_x + my_y
                res_ref[pl.ds(my_g * HG, HG)] = commxy_ref[0, :, :, 0:D]
                for j, (dx, dy) in enumerate(XY_OFFS):
                    g_j = 2 * ((my_x + dx) % 2) + ((my_y + dy) % 2)
                    res_ref[pl.ds(g_j * HG, HG)] = (
                        commxy_ref[j + 1, :, :, 0:D])
                out_ref[...] = (
                    res_ref[...].transpose(1, 0, 2).reshape(B, 1, H, D))

    return pl.pallas_call(
        body,
        grid=(NC,),
        out_shape=jax.ShapeDtypeStruct((B, 1, H, D), jnp.float32),
        in_specs=[
            pl.BlockSpec((HG, B, D), lambda c: (0, 0, 0),
                         memory_space=pltpu.VMEM),
            pl.BlockSpec((CHUNK, HG, D, nloc), lambda c: (c, 0, 0, 0),
                         memory_space=pltpu.VMEM),
            pl.BlockSpec((CHUNK, HG, D, nloc), lambda c: (c, 0, 0, 0),
                         memory_space=pltpu.VMEM),
            pl.BlockSpec((B, NBT), lambda c: (0, 0),
                         memory_space=pltpu.VMEM),
            pl.BlockSpec((B, 1), lambda c: (0, 0),
                         memory_space=pltpu.VMEM),
        ],
        out_specs=pl.BlockSpec((B, 1, H, D), lambda c: (0, 0, 0, 0),
                               memory_space=pltpu.VMEM),
        scratch_shapes=[
            pltpu.VMEM((ZDIM, HG, B, LANES), jnp.float32),
            pltpu.VMEM((4, HG, B, LANES), jnp.float32),
            pltpu.VMEM((B, nloc), jnp.float32),
            pltpu.VMEM((HG, B), jnp.float32),
            pltpu.VMEM((HG, B), jnp.float32),
            pltpu.VMEM((HG, B, D), jnp.float32),
            pltpu.VMEM((H, B, D), jnp.float32),
            pltpu.SemaphoreType.DMA((ZDIM - 1,)),
            pltpu.SemaphoreType.DMA((ZDIM - 1,)),
            pltpu.SemaphoreType.DMA((len(XY_OFFS),)),
            pltpu.SemaphoreType.DMA((len(XY_OFFS),)),
        ],
        compiler_params=pltpu.CompilerParams(
            collective_id=0,
            dimension_semantics=("arbitrary",),
            vmem_limit_bytes=100 * 1024 * 1024,
        ),
    )(qg, kg, vg, bt, lens.reshape(B, 1))


# device time: 19210 ns/iter; 1.2825x vs baseline; 1.2825x over previous
import jax
import jax.numpy as jnp
from jax import lax
from jax.experimental import pallas as pl
from jax.experimental.pallas import tpu as pltpu

ZDIM = 4
B, H, D, BS = 16, 16, 64, 16
NBT = 128
HG = H // 4
LANES = 128
NEG = -1e30
CHUNK = 4
NC = BS // CHUNK
XY_OFFS = ((0, 1), (1, 0), (1, 1))


def kernel(Q, K, V, bt, lens):
    nloc = K.shape[0]

    kt = K.transpose(1, 2, 3, 0)
    vt = V.transpose(1, 2, 3, 0)

    def _g():
        return 2 * lax.axis_index("x") + lax.axis_index("y")

    def body(q_ref, k_ref, v_ref, bt_ref, lens_ref, out_ref,
             commz_ref, commxy_ref, cnt_ref, mr_ref, lr_ref, or_ref,
             res_ref, zs_sems, zr_sems, xs_sems, xr_sems):
        c = pl.program_id(0)
        my_x = lax.axis_index("x")
        my_y = lax.axis_index("y")
        my_z = lax.axis_index("z")
        base = my_z * nloc
        barrier = pltpu.get_barrier_semaphore()

        @pl.when(c == 0)
        def _():
            with jax.named_scope("barrier_signal"):
                for dz in range(1, ZDIM):
                    pl.semaphore_signal(
                        barrier, inc=1,
                        device_id=(my_x, my_y, (my_z + dz) % ZDIM),
                        device_id_type=pl.DeviceIdType.MESH,
                    )
                for dx, dy in XY_OFFS:
                    pl.semaphore_signal(
                        barrier, inc=1,
                        device_id=((my_x + dx) % 2, (my_y + dy) % 2, my_z),
                        device_id_type=pl.DeviceIdType.MESH,
                    )
            with jax.named_scope("count"):
                btv = bt_ref[...]
                lensv = lens_ref[...]
                jidx = lax.broadcasted_iota(jnp.int32, (B, NBT, nloc), 1)
                pidx = lax.broadcasted_iota(jnp.int32, (B, NBT, nloc), 2)
                hits = ((btv[:, :, None] == base + pidx)
                        & (jidx < lensv[:, :, None]))
                cnt_ref[...] = jnp.sum(hits.astype(jnp.float32), axis=1)
            mr_ref[...] = jnp.full((HG, B), NEG, jnp.float32)
            lr_ref[...] = jnp.zeros((HG, B), jnp.float32)
            or_ref[...] = jnp.zeros((HG, B, D), jnp.float32)

        with jax.named_scope("attn_chunk"):
            cm = cnt_ref[...][None]
            my_g = 2 * my_x + my_y
            rows = lax.broadcasted_iota(jnp.int32, (HG, H), 0)
            cols = lax.broadcasted_iota(jnp.int32, (HG, H), 1)
            sel = (cols == rows + my_g * HG).astype(jnp.float32)
            q_all = lax.dot_general(
                sel, q_ref[...].reshape(B, H, D),
                (((1,), (1,)), ((), ())),
                preferred_element_type=jnp.float32,
            )
            s_list = []
            for t in range(CHUNK):
                s_t = lax.dot_general(
                    q_all, k_ref[t], (((2,), (1,)), ((0,), (0,))),
                    preferred_element_type=jnp.float32,
                ) * (D ** -0.5)
                s_list.append(jnp.where(cm > 0, s_t, NEG))
            mc = jnp.max(s_list[0], axis=2)
            for s_t in s_list[1:]:
                mc = jnp.maximum(mc, jnp.max(s_t, axis=2))
            m_old = mr_ref[...]
            m_new = jnp.maximum(m_old, mc)
            scale = jnp.exp(m_old - m_new)
            l_acc = lr_ref[...] * scale
            o_acc = or_ref[...] * scale[:, :, None]
            for t in range(CHUNK):
                e_t = jnp.exp(s_list[t] - m_new[:, :, None]) * cm
                l_acc = l_acc + jnp.sum(e_t, axis=2)
                o_acc = o_acc + lax.dot_general(
                    e_t, v_ref[t], (((2,), (2,)), ((0,), (0,))),
                    preferred_element_type=jnp.float32,
                )
            mr_ref[...] = m_new
            lr_ref[...] = l_acc
            or_ref[...] = o_acc

        @pl.when(c == NC - 1)
        def _():
            with jax.named_scope("pack_z"):
                commz_ref[0, :, :, 0:D] = or_ref[...]
                commz_ref[0, :, :, D:D + 1] = mr_ref[...][:, :, None]
                commz_ref[0, :, :, D + 1:D + 2] = lr_ref[...][:, :, None]

            with jax.named_scope("barrier_wait"):
                pl.semaphore_wait(barrier, ZDIM - 1 + len(XY_OFFS))

            with jax.named_scope("z_a2a"):
                rdmas = []
                for dz in range(1, ZDIM):
                    rdma = pltpu.make_async_remote_copy(
                        src_ref=commz_ref.at[0],
                        dst_ref=commz_ref.at[dz],
                        send_sem=zs_sems.at[dz - 1],
                        recv_sem=zr_sems.at[dz - 1],
                        device_id=(my_x, my_y, (my_z + dz) % ZDIM),
                        device_id_type=pl.DeviceIdType.MESH,
                    )
                    rdma.start()
                    rdmas.append(rdma)
                for rdma in rdmas:
                    rdma.wait()

            with jax.named_scope("merge_z"):
                ms = [commz_ref[i, :, :, D:D + 1] for i in range(ZDIM)]
                mx = ms[0]
                for mi in ms[1:]:
                    mx = jnp.maximum(mx, mi)
                acc_o = jnp.zeros((HG, B, D), jnp.float32)
                acc_l = jnp.zeros((HG, B, 1), jnp.float32)
                for i in range(ZDIM):
                    alpha = jnp.exp(ms[i] - mx)
                    acc_o = acc_o + commz_ref[i, :, :, 0:D] * alpha
                    acc_l = acc_l + commz_ref[i, :, :, D + 1:D + 2] * alpha
                commxy_ref[0, :, :, 0:D] = acc_o / acc_l

            with jax.named_scope("xy_a2a"):
                rdmas = []
                for j, (dx, dy) in enumerate(XY_OFFS):
                    rdma = pltpu.make_async_remote_copy(
                        src_ref=commxy_ref.at[0],
                        dst_ref=commxy_ref.at[j + 1],
                        send_sem=xs_sems.at[j],
                        recv_sem=xr_sems.at[j],
                        device_id=((my_x + dx) % 2, (my_y + dy) % 2, my_z),
                        device_id_type=pl.DeviceIdType.MESH,
                    )
                    rdma.start()
                    rdmas.append(rdma)
                for rdma in rdmas:
                    rdma.wait()

            with jax.named_scope("assemble"):
                my_g = 2 * my_x + my_y
                res_ref[pl.ds(my_g * HG, HG)] = commxy_ref[0, :, :, 0:D]
                for j, (dx, dy) in enumerate(XY_OFFS):
                    g_j = 2 * ((my_x + dx) % 2) + ((my_y + dy) % 2)
                    res_ref[pl.ds(g_j * HG, HG)] = (
                        commxy_ref[j + 1, :, :, 0:D])
                out_ref[...] = (
                    res_ref[...].transpose(1, 0, 2).reshape(B, 1, H, D))

    return pl.pallas_call(
        body,
        grid=(NC,),
        out_shape=jax.ShapeDtypeStruct((B, 1, H, D), jnp.float32),
        in_specs=[
            pl.BlockSpec((B, 1, H, D), lambda c: (0, 0, 0, 0),
                         memory_space=pltpu.VMEM),
            pl.BlockSpec((CHUNK, HG, D, nloc), lambda c: (c, _g(), 0, 0),
                         memory_space=pltpu.VMEM),
            pl.BlockSpec((CHUNK, HG, D, nloc), lambda c: (c, _g(), 0, 0),
                         memory_space=pltpu.VMEM),
            pl.BlockSpec((B, NBT), lambda c: (0, 0),
                         memory_space=pltpu.VMEM),
            pl.BlockSpec((B, 1), lambda c: (0, 0),
                         memory_space=pltpu.VMEM),
        ],
        out_specs=pl.BlockSpec((B, 1, H, D), lambda c: (0, 0, 0, 0),
                               memory_space=pltpu.VMEM),
        scratch_shapes=[
            pltpu.VMEM((ZDIM, HG, B, LANES), jnp.float32),
            pltpu.VMEM((4, HG, B, LANES), jnp.float32),
            pltpu.VMEM((B, nloc), jnp.float32),
            pltpu.VMEM((HG, B), jnp.float32),
            pltpu.VMEM((HG, B), jnp.float32),
            pltpu.VMEM((HG, B, D), jnp.float32),
            pltpu.VMEM((H, B, D), jnp.float32),
            pltpu.SemaphoreType.DMA((ZDIM - 1,)),
            pltpu.SemaphoreType.DMA((ZDIM - 1,)),
            pltpu.SemaphoreType.DMA((len(XY_OFFS),)),
            pltpu.SemaphoreType.DMA((len(XY_OFFS),)),
        ],
        compiler_params=pltpu.CompilerParams(
            collective_id=0,
            dimension_semantics=("arbitrary",),
            vmem_limit_bytes=100 * 1024 * 1024,
        ),
    )(Q, kt, vt, bt, lens.reshape(B, 1))


# device time: 17956 ns/iter; 1.3720x vs baseline; 1.0698x over previous
import jax
import jax.numpy as jnp
from jax import lax
from jax.experimental import pallas as pl
from jax.experimental.pallas import tpu as pltpu

ZDIM = 4
B, H, D, BS = 16, 16, 64, 16
NBT = 128
HG = H // 4
LANES = 128
NEG = -1e30
CHUNK = 8
NC = BS // CHUNK
XY_OFFS = ((0, 1), (1, 0), (1, 1))


def kernel(Q, K, V, bt, lens):
    nloc = K.shape[0]

    kt = K.transpose(1, 2, 3, 0)
    vt = V.transpose(1, 2, 3, 0)

    def _g():
        return 2 * lax.axis_index("x") + lax.axis_index("y")

    def body(q_ref, k_ref, v_ref, bt_ref, lens_ref, out_ref,
             commz_ref, commxy_ref, cnt_ref, mr_ref, lr_ref, or_ref,
             res_ref, zs_sems, zr_sems, xs_sems, xr_sems):
        c = pl.program_id(0)
        my_x = lax.axis_index("x")
        my_y = lax.axis_index("y")
        my_z = lax.axis_index("z")
        base = my_z * nloc
        barrier = pltpu.get_barrier_semaphore()

        @pl.when(c == 0)
        def _():
            with jax.named_scope("barrier_signal"):
                for dz in range(1, ZDIM):
                    pl.semaphore_signal(
                        barrier, inc=1,
                        device_id=(my_x, my_y, (my_z + dz) % ZDIM),
                        device_id_type=pl.DeviceIdType.MESH,
                    )
                for dx, dy in XY_OFFS:
                    pl.semaphore_signal(
                        barrier, inc=1,
                        device_id=((my_x + dx) % 2, (my_y + dy) % 2, my_z),
                        device_id_type=pl.DeviceIdType.MESH,
                    )
            with jax.named_scope("count"):
                btv = bt_ref[...]
                lensv = lens_ref[...]
                jidx = lax.broadcasted_iota(jnp.int32, (B, NBT, nloc), 1)
                pidx = lax.broadcasted_iota(jnp.int32, (B, NBT, nloc), 2)
                hits = ((btv[:, :, None] == base + pidx)
                        & (jidx < lensv[:, :, None]))
                cnt_ref[...] = jnp.sum(hits.astype(jnp.float32), axis=1)
            mr_ref[...] = jnp.full((HG, B), NEG, jnp.float32)
            lr_ref[...] = jnp.zeros((HG, B), jnp.float32)
            or_ref[...] = jnp.zeros((HG, B, D), jnp.float32)

        with jax.named_scope("attn_chunk"):
            cm = cnt_ref[...][None]
            my_g = 2 * my_x + my_y
            rows = lax.broadcasted_iota(jnp.int32, (HG, H), 0)
            cols = lax.broadcasted_iota(jnp.int32, (HG, H), 1)
            sel = (cols == rows + my_g * HG).astype(jnp.float32)
            q_all = lax.dot_general(
                sel, q_ref[...].reshape(B, H, D),
                (((1,), (1,)), ((), ())),
                preferred_element_type=jnp.float32,
            )
            s_list = []
            for t in range(CHUNK):
                s_t = lax.dot_general(
                    q_all, k_ref[t], (((2,), (1,)), ((0,), (0,))),
                    preferred_element_type=jnp.float32,
                ) * (D ** -0.5)
                s_list.append(jnp.where(cm > 0, s_t, NEG))
            mc = jnp.max(s_list[0], axis=2)
            for s_t in s_list[1:]:
                mc = jnp.maximum(mc, jnp.max(s_t, axis=2))
            m_old = mr_ref[...]
            m_new = jnp.maximum(m_old, mc)
            scale = jnp.exp(m_old - m_new)
            l_acc = lr_ref[...] * scale
            o_acc = or_ref[...] * scale[:, :, None]
            for t in range(CHUNK):
                e_t = jnp.exp(s_list[t] - m_new[:, :, None]) * cm
                l_acc = l_acc + jnp.sum(e_t, axis=2)
                o_acc = o_acc + lax.dot_general(
                    e_t, v_ref[t], (((2,), (2,)), ((0,), (0,))),
                    preferred_element_type=jnp.float32,
                )
            mr_ref[...] = m_new
            lr_ref[...] = l_acc
            or_ref[...] = o_acc

        @pl.when(c == NC - 1)
        def _():
            with jax.named_scope("pack_z"):
                commz_ref[0, :, :, 0:D] = or_ref[...]
                commz_ref[0, :, :, D:D + 1] = mr_ref[...][:, :, None]
                commz_ref[0, :, :, D + 1:D + 2] = lr_ref[...][:, :, None]

            with jax.named_scope("barrier_wait"):
                pl.semaphore_wait(barrier, ZDIM - 1 + len(XY_OFFS))

            with jax.named_scope("z_a2a"):
                rdmas = []
                for dz in range(1, ZDIM):
                    rdma = pltpu.make_async_remote_copy(
                        src_ref=commz_ref.at[0],
                        dst_ref=commz_ref.at[dz],
                        send_sem=zs_sems.at[dz - 1],
                        recv_sem=zr_sems.at[dz - 1],
                        device_id=(my_x, my_y, (my_z + dz) % ZDIM),
                        device_id_type=pl.DeviceIdType.MESH,
                    )
                    rdma.start()
                    rdmas.append(rdma)
                for rdma in rdmas:
                    rdma.wait()

            with jax.named_scope("merge_z"):
                ms = [commz_ref[i, :, :, D:D + 1] for i in range(ZDIM)]
                mx = ms[0]
                for mi in ms[1:]:
                    mx = jnp.maximum(mx, mi)
                acc_o = jnp.zeros((HG, B, D), jnp.float32)
                acc_l = jnp.zeros((HG, B, 1), jnp.float32)
                for i in range(ZDIM):
                    alpha = jnp.exp(ms[i] - mx)
                    acc_o = acc_o + commz_ref[i, :, :, 0:D] * alpha
                    acc_l = acc_l + commz_ref[i, :, :, D + 1:D + 2] * alpha
                commxy_ref[0, :, :, 0:D] = acc_o / acc_l

            with jax.named_scope("xy_a2a"):
                rdmas = []
                for j, (dx, dy) in enumerate(XY_OFFS):
                    rdma = pltpu.make_async_remote_copy(
                        src_ref=commxy_ref.at[0],
                        dst_ref=commxy_ref.at[j + 1],
                        send_sem=xs_sems.at[j],
                        recv_sem=xr_sems.at[j],
                        device_id=((my_x + dx) % 2, (my_y + dy) % 2, my_z),
                        device_id_type=pl.DeviceIdType.MESH,
                    )
                    rdma.start()
                    rdmas.append(rdma)
                for rdma in rdmas:
                    rdma.wait()

            with jax.named_scope("assemble"):
                my_g = 2 * my_x + my_y
                res_ref[pl.ds(my_g * HG, HG)] = commxy_ref[0, :, :, 0:D]
                for j, (dx, dy) in enumerate(XY_OFFS):
                    g_j = 2 * ((my_x + dx) % 2) + ((my_y + dy) % 2)
                    res_ref[pl.ds(g_j * HG, HG)] = (
                        commxy_ref[j + 1, :, :, 0:D])
                out_ref[...] = (
                    res_ref[...].transpose(1, 0, 2).reshape(B, 1, H, D))

    return pl.pallas_call(
        body,
        grid=(NC,),
        out_shape=jax.ShapeDtypeStruct((B, 1, H, D), jnp.float32),
        in_specs=[
            pl.BlockSpec((B, 1, H, D), lambda c: (0, 0, 0, 0),
                         memory_space=pltpu.VMEM),
            pl.BlockSpec((CHUNK, HG, D, nloc), lambda c: (c, _g(), 0, 0),
                         memory_space=pltpu.VMEM),
            pl.BlockSpec((CHUNK, HG, D, nloc), lambda c: (c, _g(), 0, 0),
                         memory_space=pltpu.VMEM),
            pl.BlockSpec((B, NBT), lambda c: (0, 0),
                         memory_space=pltpu.VMEM),
            pl.BlockSpec((B, 1), lambda c: (0, 0),
                         memory_space=pltpu.VMEM),
        ],
        out_specs=pl.BlockSpec((B, 1, H, D), lambda c: (0, 0, 0, 0),
                               memory_space=pltpu.VMEM),
        scratch_shapes=[
            pltpu.VMEM((ZDIM, HG, B, LANES), jnp.float32),
            pltpu.VMEM((4, HG, B, LANES), jnp.float32),
            pltpu.VMEM((B, nloc), jnp.float32),
            pltpu.VMEM((HG, B), jnp.float32),
            pltpu.VMEM((HG, B), jnp.float32),
            pltpu.VMEM((HG, B, D), jnp.float32),
            pltpu.VMEM((H, B, D), jnp.float32),
            pltpu.SemaphoreType.DMA((ZDIM - 1,)),
            pltpu.SemaphoreType.DMA((ZDIM - 1,)),
            pltpu.SemaphoreType.DMA((len(XY_OFFS),)),
            pltpu.SemaphoreType.DMA((len(XY_OFFS),)),
        ],
        compiler_params=pltpu.CompilerParams(
            collective_id=0,
            dimension_semantics=("arbitrary",),
            vmem_limit_bytes=100 * 1024 * 1024,
        ),
    )(Q, kt, vt, bt, lens.reshape(B, 1))


# device time: 17320 ns/iter; 1.4224x vs baseline; 1.0367x over previous
import jax
import jax.numpy as jnp
from jax import lax
from jax.experimental import pallas as pl
from jax.experimental.pallas import tpu as pltpu

ZDIM = 4
B, H, D, BS = 16, 16, 64, 16
NBT = 128
HG = H // 4
LANES = 128
NEG = -1e30
CHUNK = 16
NC = BS // CHUNK
XY_OFFS = ((0, 1), (1, 0), (1, 1))


def kernel(Q, K, V, bt, lens):
    nloc = K.shape[0]

    kt = K.transpose(1, 2, 3, 0)
    vt = V.transpose(1, 2, 3, 0)

    def _g():
        return 2 * lax.axis_index("x") + lax.axis_index("y")

    def body(q_ref, k_ref, v_ref, bt_ref, lens_ref, out_ref,
             commz_ref, commxy_ref, cnt_ref, mr_ref, lr_ref, or_ref,
             res_ref, zs_sems, zr_sems, xs_sems, xr_sems):
        c = pl.program_id(0)
        my_x = lax.axis_index("x")
        my_y = lax.axis_index("y")
        my_z = lax.axis_index("z")
        base = my_z * nloc
        barrier = pltpu.get_barrier_semaphore()

        @pl.when(c == 0)
        def _():
            with jax.named_scope("barrier_signal"):
                for dz in range(1, ZDIM):
                    pl.semaphore_signal(
                        barrier, inc=1,
                        device_id=(my_x, my_y, (my_z + dz) % ZDIM),
                        device_id_type=pl.DeviceIdType.MESH,
                    )
                for dx, dy in XY_OFFS:
                    pl.semaphore_signal(
                        barrier, inc=1,
                        device_id=((my_x + dx) % 2, (my_y + dy) % 2, my_z),
                        device_id_type=pl.DeviceIdType.MESH,
                    )
            with jax.named_scope("count"):
                btv = bt_ref[...]
                lensv = lens_ref[...]
                jidx = lax.broadcasted_iota(jnp.int32, (B, NBT, nloc), 1)
                pidx = lax.broadcasted_iota(jnp.int32, (B, NBT, nloc), 2)
                hits = ((btv[:, :, None] == base + pidx)
                        & (jidx < lensv[:, :, None]))
                cnt_ref[...] = jnp.sum(hits.astype(jnp.float32), axis=1)
            mr_ref[...] = jnp.full((HG, B), NEG, jnp.float32)
            lr_ref[...] = jnp.zeros((HG, B), jnp.float32)
            or_ref[...] = jnp.zeros((HG, B, D), jnp.float32)

        with jax.named_scope("attn_chunk"):
            cm = cnt_ref[...][None]
            my_g = 2 * my_x + my_y
            rows = lax.broadcasted_iota(jnp.int32, (HG, H), 0)
            cols = lax.broadcasted_iota(jnp.int32, (HG, H), 1)
            sel = (cols == rows + my_g * HG).astype(jnp.float32)
            q_all = lax.dot_general(
                sel, q_ref[...].reshape(B, H, D),
                (((1,), (1,)), ((), ())),
                preferred_element_type=jnp.float32,
            )
            s_list = []
            for t in range(CHUNK):
                s_t = lax.dot_general(
                    q_all, k_ref[t], (((2,), (1,)), ((0,), (0,))),
                    preferred_element_type=jnp.float32,
                ) * (D ** -0.5)
                s_list.append(jnp.where(cm > 0, s_t, NEG))
            mc = jnp.max(s_list[0], axis=2)
            for s_t in s_list[1:]:
                mc = jnp.maximum(mc, jnp.max(s_t, axis=2))
            m_old = mr_ref[...]
            m_new = jnp.maximum(m_old, mc)
            scale = jnp.exp(m_old - m_new)
            l_acc = lr_ref[...] * scale
            o_acc = or_ref[...] * scale[:, :, None]
            for t in range(CHUNK):
                e_t = jnp.exp(s_list[t] - m_new[:, :, None]) * cm
                l_acc = l_acc + jnp.sum(e_t, axis=2)
                o_acc = o_acc + lax.dot_general(
                    e_t, v_ref[t], (((2,), (2,)), ((0,), (0,))),
                    preferred_element_type=jnp.float32,
                )
            mr_ref[...] = m_new
            lr_ref[...] = l_acc
            or_ref[...] = o_acc

        @pl.when(c == NC - 1)
        def _():
            with jax.named_scope("pack_z"):
                commz_ref[0, :, :, 0:D] = or_ref[...]
                commz_ref[0, :, :, D:D + 1] = mr_ref[...][:, :, None]
                commz_ref[0, :, :, D + 1:D + 2] = lr_ref[...][:, :, None]

            with jax.named_scope("barrier_wait"):
                pl.semaphore_wait(barrier, ZDIM - 1 + len(XY_OFFS))

            with jax.named_scope("z_a2a"):
                rdmas = []
                for dz in range(1, ZDIM):
                    rdma = pltpu.make_async_remote_copy(
                        src_ref=commz_ref.at[0],
                        dst_ref=commz_ref.at[dz],
                        send_sem=zs_sems.at[dz - 1],
                        recv_sem=zr_sems.at[dz - 1],
                        device_id=(my_x, my_y, (my_z + dz) % ZDIM),
                        device_id_type=pl.DeviceIdType.MESH,
                    )
                    rdma.start()
                    rdmas.append(rdma)
                for rdma in rdmas:
                    rdma.wait()

            with jax.named_scope("merge_z"):
                ms = [commz_ref[i, :, :, D:D + 1] for i in range(ZDIM)]
                mx = ms[0]
                for mi in ms[1:]:
                    mx = jnp.maximum(mx, mi)
                acc_o = jnp.zeros((HG, B, D), jnp.float32)
                acc_l = jnp.zeros((HG, B, 1), jnp.float32)
                for i in range(ZDIM):
                    alpha = jnp.exp(ms[i] - mx)
                    acc_o = acc_o + commz_ref[i, :, :, 0:D] * alpha
                    acc_l = acc_l + commz_ref[i, :, :, D + 1:D + 2] * alpha
                commxy_ref[0, :, :, 0:D] = acc_o / acc_l

            with jax.named_scope("xy_a2a"):
                rdmas = []
                for j, (dx, dy) in enumerate(XY_OFFS):
                    rdma = pltpu.make_async_remote_copy(
                        src_ref=commxy_ref.at[0],
                        dst_ref=commxy_ref.at[j + 1],
                        send_sem=xs_sems.at[j],
                        recv_sem=xr_sems.at[j],
                        device_id=((my_x + dx) % 2, (my_y + dy) % 2, my_z),
                        device_id_type=pl.DeviceIdType.MESH,
                    )
                    rdma.start()
                    rdmas.append(rdma)
                for rdma in rdmas:
                    rdma.wait()

            with jax.named_scope("assemble"):
                my_g = 2 * my_x + my_y
                res_ref[pl.ds(my_g * HG, HG)] = commxy_ref[0, :, :, 0:D]
                for j, (dx, dy) in enumerate(XY_OFFS):
                    g_j = 2 * ((my_x + dx) % 2) + ((my_y + dy) % 2)
                    res_ref[pl.ds(g_j * HG, HG)] = (
                        commxy_ref[j + 1, :, :, 0:D])
                out_ref[...] = (
                    res_ref[...].transpose(1, 0, 2).reshape(B, 1, H, D))

    return pl.pallas_call(
        body,
        grid=(NC,),
        out_shape=jax.ShapeDtypeStruct((B, 1, H, D), jnp.float32),
        in_specs=[
            pl.BlockSpec((B, 1, H, D), lambda c: (0, 0, 0, 0),
                         memory_space=pltpu.VMEM),
            pl.BlockSpec((CHUNK, HG, D, nloc), lambda c: (c, _g(), 0, 0),
                         memory_space=pltpu.VMEM),
            pl.BlockSpec((CHUNK, HG, D, nloc), lambda c: (c, _g(), 0, 0),
                         memory_space=pltpu.VMEM),
            pl.BlockSpec((B, NBT), lambda c: (0, 0),
                         memory_space=pltpu.VMEM),
            pl.BlockSpec((B, 1), lambda c: (0, 0),
                         memory_space=pltpu.VMEM),
        ],
        out_specs=pl.BlockSpec((B, 1, H, D), lambda c: (0, 0, 0, 0),
                               memory_space=pltpu.VMEM),
        scratch_shapes=[
            pltpu.VMEM((ZDIM, HG, B, LANES), jnp.float32),
            pltpu.VMEM((4, HG, B, LANES), jnp.float32),
            pltpu.VMEM((B, nloc), jnp.float32),
            pltpu.VMEM((HG, B), jnp.float32),
            pltpu.VMEM((HG, B), jnp.float32),
            pltpu.VMEM((HG, B, D), jnp.float32),
            pltpu.VMEM((H, B, D), jnp.float32),
            pltpu.SemaphoreType.DMA((ZDIM - 1,)),
            pltpu.SemaphoreType.DMA((ZDIM - 1,)),
            pltpu.SemaphoreType.DMA((len(XY_OFFS),)),
            pltpu.SemaphoreType.DMA((len(XY_OFFS),)),
        ],
        compiler_params=pltpu.CompilerParams(
            collective_id=0,
            dimension_semantics=("arbitrary",),
            vmem_limit_bytes=100 * 1024 * 1024,
        ),
    )(Q, kt, vt, bt, lens.reshape(B, 1))
